# Initial kernel scaffold; baseline (speedup 1.0000x reference)
import jax
import jax.numpy as jnp
from jax.experimental import pallas as pl


def kernel(x, tok_table, pos_table):
    raise NotImplementedError("write your pallas kernel here")


# speedup vs baseline: 1.2798x; 1.2798x over previous
import functools

import jax
import jax.numpy as jnp
from jax import lax
from jax.experimental import pallas as pl
from jax.experimental.pallas import tpu as pltpu
from jax.experimental.pallas import tpu_sc as plsc

NC = 2
NS = 16
LANES = 16

B = 4
T = 2048
D = 128
ROWS = B * T
NW = NC * NS
BPW = ROWS // NW
IDX_CHUNK = 128
NCHUNK = BPW // IDX_CHUNK


def _body(tok_hbm, x_hbm, pos_hbm, out_hbm, idx_v, tok_v, pos_v, sem):
    wid = lax.axis_index("s") * NC + lax.axis_index("c")
    base = wid * BPW
    pltpu.sync_copy(x_hbm.at[pl.ds(NCHUNK * wid, NCHUNK)], idx_v)

    cps = []
    for k in range(NCHUNK):
        cps.append(
            pltpu.async_copy(
                tok_hbm.at[idx_v.at[k]],
                tok_v.at[pl.ds(k * IDX_CHUNK, IDX_CHUNK)],
                sem,
            )
        )
    pos_base = lax.rem(base, T)
    pltpu.sync_copy(pos_hbm.at[pl.ds(pos_base, BPW)], pos_v)
    for cp in cps:
        cp.wait()

    def add_row(i, carry):
        for j in range(D // LANES):
            sl = pl.ds(j * LANES, LANES)
            tok_v[i, sl] = tok_v[i, sl] + pos_v[i, sl]
        return carry

    lax.fori_loop(0, BPW, add_row, 0)

    pltpu.sync_copy(tok_v, out_hbm.at[pl.ds(base, BPW)])


@functools.partial(jax.jit, donate_argnums=())
def kernel(x, tok_table, pos_table):
    x2 = x.reshape(ROWS // IDX_CHUNK, IDX_CHUNK)
    mesh = plsc.VectorSubcoreMesh(
        core_axis_name="c", subcore_axis_name="s",
        num_cores=NC, num_subcores=NS,
    )
    run = pl.kernel(
        _body,
        out_type=jax.ShapeDtypeStruct((ROWS, D), jnp.float32),
        mesh=mesh,
        scratch_types=[
            pltpu.VMEM((NCHUNK, IDX_CHUNK), jnp.int32),
            pltpu.VMEM((BPW, D), jnp.float32),
            pltpu.VMEM((BPW, D), jnp.float32),
            pltpu.SemaphoreType.DMA,
        ],
    )
    out = run(tok_table, x2, pos_table)
    return out.reshape(B, T, D)


# speedup vs baseline: 1.2965x; 1.0130x over previous
import functools

import jax
import jax.numpy as jnp
from jax import lax
from jax.experimental import pallas as pl
from jax.experimental.pallas import tpu as pltpu
from jax.experimental.pallas import tpu_sc as plsc

NC = 2
NS = 16
LANES = 16

B = 4
T = 2048
D = 128
NW = NC * NS
TPW = T // NW
ROWS_PW = B * TPW


def _body(tok_hbm, x_hbm, pos_hbm, out_hbm, idx_v, tok_v, pos_v, sem):
    wid = lax.axis_index("s") * NC + lax.axis_index("c")
    p0 = wid * TPW

    for b in range(B):
        pltpu.sync_copy(x_hbm.at[b, pl.ds(p0, TPW)], idx_v.at[b])

    cps = [
        pltpu.async_copy(
            tok_hbm.at[idx_v.at[b]],
            tok_v.at[pl.ds(b * TPW, TPW)],
            sem,
        )
        for b in range(B)
    ]
    pltpu.sync_copy(pos_hbm.at[pl.ds(p0, TPW)], pos_v)
    for cp in cps:
        cp.wait()

    def add_row(t, carry):
        for j in range(D // LANES):
            sl = pl.ds(j * LANES, LANES)
            p = pos_v[t, sl]
            for b in range(B):
                tok_v[b * TPW + t, sl] = tok_v[b * TPW + t, sl] + p
        return carry

    lax.fori_loop(0, TPW, add_row, 0)

    for b in range(B):
        pltpu.sync_copy(
            tok_v.at[pl.ds(b * TPW, TPW)],
            out_hbm.at[b, pl.ds(p0, TPW)],
        )


@jax.jit
def kernel(x, tok_table, pos_table):
    mesh = plsc.VectorSubcoreMesh(
        core_axis_name="c", subcore_axis_name="s",
        num_cores=NC, num_subcores=NS,
    )
    run = pl.kernel(
        _body,
        out_type=jax.ShapeDtypeStruct((B, T, D), jnp.float32),
        mesh=mesh,
        scratch_types=[
            pltpu.VMEM((B, TPW), jnp.int32),
            pltpu.VMEM((ROWS_PW, D), jnp.float32),
            pltpu.VMEM((TPW, D), jnp.float32),
            pltpu.SemaphoreType.DMA,
        ],
    )
    return run(tok_table, x, pos_table)
